# baseline (speedup 1.0000x reference)
import functools

import jax
import jax.numpy as jnp
from jax import lax
from jax.experimental import pallas as pl
from jax.experimental.pallas import tpu as pltpu
from jax.experimental.pallas import tpu_sc as plsc

B = 1
T = 2048
D = 768
E = 8
K = 2
F = 3072
FC = 768

A = T * K
BLK = 128
P = 5120
NT = P // BLK
NC, NS, L = 2, 16, 16
NW = NC * NS
ROWS_G = P // NW
GCH = 10
GRC = ROWS_G // GCH
TOK_C = T // NW
NEG = -1e30


def _router_body(x_ref, wg_ref, tw_ref, ti_ref, aux_ref):
    x = x_ref[...]
    wg = wg_ref[...]
    logits = lax.dot_general(x, wg, (((1,), (1,)), ((), ())),
                             preferred_element_type=jnp.float32)
    col = lax.broadcasted_iota(jnp.int32, (T, 128), 1)
    lg = jnp.where(col < E, logits, NEG)
    m1 = jnp.max(lg, axis=1, keepdims=True)
    i1 = jnp.min(jnp.where(lg == m1, col, 128), axis=1, keepdims=True)
    lg2 = jnp.where(col == i1, NEG, lg)
    m2 = jnp.max(lg2, axis=1, keepdims=True)
    i2 = jnp.min(jnp.where(lg2 == m2, col, 128), axis=1, keepdims=True)
    s = jnp.exp(m2 - m1)
    w1 = 1.0 / (1.0 + s)
    tw_ref[...] = jnp.concatenate([w1, s * w1], axis=1)
    ti_ref[...] = jnp.concatenate([i1, i2], axis=1)
    probs = jnp.exp(lg - m1)
    probs = probs / jnp.sum(probs, axis=1, keepdims=True)
    mean_p = jnp.sum(probs, axis=0) * jnp.float32(1.0 / T)
    aux_ref[0, 0] = jnp.sum(mean_p * mean_p) * jnp.float32(E)


def _router(flat_x, wg_pad):
    return pl.pallas_call(
        _router_body,
        out_shape=(
            jax.ShapeDtypeStruct((T, K), jnp.float32),
            jax.ShapeDtypeStruct((T, K), jnp.int32),
            jax.ShapeDtypeStruct((1, 1), jnp.float32),
        ),
        out_specs=(
            pl.BlockSpec(memory_space=pltpu.VMEM),
            pl.BlockSpec(memory_space=pltpu.VMEM),
            pl.BlockSpec(memory_space=pltpu.SMEM),
        ),
    )(flat_x, wg_pad)


def _dispatch_meta(ti, tw):
    e_flat = ti.reshape(-1)
    oh = (e_flat[:, None] == jnp.arange(E, dtype=jnp.int32)[None, :])
    cum = jnp.cumsum(oh.astype(jnp.int32), axis=0)
    counts = cum[-1]
    rank = jnp.take_along_axis(cum, e_flat[:, None], axis=1)[:, 0] - 1
    pc = ((counts + BLK - 1) // BLK) * BLK
    cpc = jnp.cumsum(pc)
    off = cpc - pc
    p_a = (off[e_flat] + rank).astype(jnp.int32)
    tok = (jnp.arange(A, dtype=jnp.int32) // K)
    packed = jnp.stack([tok.astype(jnp.float32), tw.reshape(-1)], axis=1)
    row_tw = jnp.zeros((P, 2), jnp.float32).at[p_a].set(packed)
    row_token = row_tw[:, 0].astype(jnp.int32)
    row_w = row_tw[:, 1]
    pos = p_a.reshape(T, K)
    total = cpc[-1]
    starts = jnp.arange(NT, dtype=jnp.int32) * BLK
    tile_e = jnp.minimum(
        jnp.sum((cpc[None, :] <= starts[:, None]).astype(jnp.int32), axis=1),
        E - 1)
    tile_used = (starts < total).astype(jnp.int32)
    return row_token, row_w, pos, tile_e, tile_used


def _sc_mesh():
    return plsc.VectorSubcoreMesh(core_axis_name="c", subcore_axis_name="s",
                                  num_cores=NC)


@functools.cache
def _build_sc_gather():
    @functools.partial(
        pl.kernel,
        out_type=jax.ShapeDtypeStruct((P, D), jnp.float32),
        mesh=_sc_mesh(),
        scratch_types=[
            pltpu.VMEM((ROWS_G,), jnp.int32),
            pltpu.VMEM((ROWS_G, D), jnp.float32),
            pltpu.SemaphoreType.DMA,
        ],
    )
    def _sc_gather(x_hbm, idx_hbm, out_hbm, idx_v, rows_v, sem):
        wid = lax.axis_index("s") * NC + lax.axis_index("c")
        base = wid * ROWS_G
        pltpu.sync_copy(idx_hbm.at[pl.ds(base, ROWS_G)], idx_v)
        copies = []
        for c in range(GCH):
            sl = pl.ds(c * GRC, GRC)
            copies.append(
                pltpu.async_copy(x_hbm.at[idx_v.at[sl]], rows_v.at[sl], sem))
        for cp in copies:
            cp.wait()
        pltpu.sync_copy(rows_v, out_hbm.at[pl.ds(base, ROWS_G)])

    return _sc_gather


def _ffn_body(te_ref, used_ref, xs_ref, w1_ref, w2_ref, rw_ref, y_ref,
              w1b_ref, w2b_ref):
    j = pl.program_id(0)
    jm1 = jnp.maximum(j - 1, 0)
    changed = jnp.logical_or(j == 0, te_ref[j] != te_ref[jm1])

    @pl.when(changed)
    def _():
        w1b_ref[...] = w1_ref[0].astype(jnp.bfloat16)
        w2b_ref[...] = w2_ref[0].astype(jnp.bfloat16)

    @pl.when(used_ref[j] == 1)
    def _():
        xb = xs_ref[...].astype(jnp.bfloat16)
        y = jnp.zeros((BLK, D), jnp.float32)
        for f in range(F // FC):
            w1c = w1b_ref[:, f * FC:(f + 1) * FC]
            h = jnp.dot(xb, w1c, preferred_element_type=jnp.float32)
            h = jax.nn.gelu(h).astype(jnp.bfloat16)
            w2c = w2b_ref[f * FC:(f + 1) * FC, :]
            y = y + jnp.dot(h, w2c, preferred_element_type=jnp.float32)
        y_ref[...] = y * rw_ref[:, 0:1]

    @pl.when(used_ref[j] == 0)
    def _():
        y_ref[...] = jnp.zeros((BLK, D), jnp.float32)


def _ffn(tile_e, tile_used, x_sorted, w1, w2, rw_b):
    grid_spec = pltpu.PrefetchScalarGridSpec(
        num_scalar_prefetch=2,
        grid=(NT,),
        in_specs=[
            pl.BlockSpec((BLK, D), lambda j, te, us: (j, 0)),
            pl.BlockSpec((1, D, F), lambda j, te, us: (te[j], 0, 0)),
            pl.BlockSpec((1, F, D), lambda j, te, us: (te[j], 0, 0)),
            pl.BlockSpec((BLK, 128), lambda j, te, us: (j, 0)),
        ],
        out_specs=pl.BlockSpec((BLK, D), lambda j, te, us: (j, 0)),
        scratch_shapes=[
            pltpu.VMEM((D, F), jnp.bfloat16),
            pltpu.VMEM((F, D), jnp.bfloat16),
        ],
    )
    return pl.pallas_call(
        _ffn_body,
        grid_spec=grid_spec,
        out_shape=jax.ShapeDtypeStruct((P, D), jnp.float32),
        compiler_params=pltpu.CompilerParams(
            dimension_semantics=("arbitrary",)),
    )(tile_e, tile_used, x_sorted, w1, w2, rw_b)


@functools.cache
def _build_sc_combine():
    @functools.partial(
        pl.kernel,
        out_type=jax.ShapeDtypeStruct((T, D), jnp.float32),
        mesh=_sc_mesh(),
        scratch_types=[
            pltpu.VMEM((TOK_C,), jnp.int32),
            pltpu.VMEM((TOK_C,), jnp.int32),
            pltpu.VMEM((TOK_C, D), jnp.float32),
            pltpu.VMEM((TOK_C, D), jnp.float32),
            pltpu.SemaphoreType.DMA,
        ],
    )
    def _sc_combine(y_hbm, pos0_hbm, pos1_hbm, out_hbm,
                    i0_v, i1_v, b0, b1, sem):
        wid = lax.axis_index("s") * NC + lax.axis_index("c")
        base = wid * TOK_C
        pltpu.sync_copy(pos0_hbm.at[pl.ds(base, TOK_C)], i0_v)
        pltpu.sync_copy(pos1_hbm.at[pl.ds(base, TOK_C)], i1_v)
        c0 = pltpu.async_copy(y_hbm.at[i0_v], b0, sem)
        c1 = pltpu.async_copy(y_hbm.at[i1_v], b1, sem)
        c0.wait()
        c1.wait()

        def row_body(r, carry):
            for c in range(D // L):
                sl = pl.ds(c * L, L)
                b0[r, sl] = b0[r, sl] + b1[r, sl]
            return carry

        lax.fori_loop(0, TOK_C, row_body, 0)
        pltpu.sync_copy(b0, out_hbm.at[pl.ds(base, TOK_C)])

    return _sc_combine


def kernel(x, Wg, W1, W2):
    flat_x = x.reshape(T, D)
    wg_pad = jnp.zeros((128, D), jnp.float32).at[:E].set(Wg)
    tw, ti, aux = _router(flat_x, wg_pad)
    row_token, row_w, pos, tile_e, tile_used = _dispatch_meta(ti, tw)
    rw_b = jnp.broadcast_to(row_w[:, None], (P, 128))
    x_sorted = _build_sc_gather()(flat_x, row_token)
    y = _ffn(tile_e, tile_used, x_sorted, W1, W2, rw_b)
    out = _build_sc_combine()(y, pos[:, 0], pos[:, 1])
    return out.reshape(B, T, D), aux.reshape(())


# speedup vs baseline: 1.0236x; 1.0236x over previous
import functools

import jax
import jax.numpy as jnp
from jax import lax
from jax.experimental import pallas as pl
from jax.experimental.pallas import tpu as pltpu
from jax.experimental.pallas import tpu_sc as plsc

B = 1
T = 2048
D = 768
E = 8
K = 2
F = 3072
FC = 768

A = T * K
BLK = 128
P = 5120
NT = P // BLK
NC, NS, L = 2, 16, 16
NW = NC * NS
ROWS_G = P // NW
GCH = 10
GRC = ROWS_G // GCH
TOK_C = T // NW
NEG = -1e30


def _router_body(x_ref, wg_ref, tw_ref, ti_ref, aux_ref):
    x = x_ref[...]
    wg = wg_ref[...]
    logits = lax.dot_general(x, wg, (((1,), (1,)), ((), ())),
                             preferred_element_type=jnp.float32)
    col = lax.broadcasted_iota(jnp.int32, (T, 128), 1)
    lg = jnp.where(col < E, logits, NEG)
    m1 = jnp.max(lg, axis=1, keepdims=True)
    i1 = jnp.min(jnp.where(lg == m1, col, 128), axis=1, keepdims=True)
    lg2 = jnp.where(col == i1, NEG, lg)
    m2 = jnp.max(lg2, axis=1, keepdims=True)
    i2 = jnp.min(jnp.where(lg2 == m2, col, 128), axis=1, keepdims=True)
    s = jnp.exp(m2 - m1)
    w1 = 1.0 / (1.0 + s)
    tw_ref[...] = jnp.concatenate([w1, s * w1], axis=1)
    ti_ref[...] = jnp.concatenate([i1, i2], axis=1)
    probs = jnp.exp(lg - m1)
    probs = probs / jnp.sum(probs, axis=1, keepdims=True)
    mean_p = jnp.sum(probs, axis=0) * jnp.float32(1.0 / T)
    aux_ref[0, 0] = jnp.sum(mean_p * mean_p) * jnp.float32(E)


def _router(flat_x, wg_pad):
    return pl.pallas_call(
        _router_body,
        out_shape=(
            jax.ShapeDtypeStruct((T, K), jnp.float32),
            jax.ShapeDtypeStruct((T, K), jnp.int32),
            jax.ShapeDtypeStruct((1, 1), jnp.float32),
        ),
        out_specs=(
            pl.BlockSpec(memory_space=pltpu.VMEM),
            pl.BlockSpec(memory_space=pltpu.VMEM),
            pl.BlockSpec(memory_space=pltpu.SMEM),
        ),
    )(flat_x, wg_pad)


def _dispatch_meta(ti, tw):
    e_flat = ti.reshape(-1)
    oh = (e_flat[:, None] == jnp.arange(E, dtype=jnp.int32)[None, :])
    cum = jnp.cumsum(oh.astype(jnp.int32), axis=0)
    counts = cum[-1]
    rank = jnp.take_along_axis(cum, e_flat[:, None], axis=1)[:, 0] - 1
    pc = ((counts + BLK - 1) // BLK) * BLK
    cpc = jnp.cumsum(pc)
    off = cpc - pc
    p_a = (off[e_flat] + rank).astype(jnp.int32)
    tok = (jnp.arange(A, dtype=jnp.int32) // K)
    row_token = jnp.zeros((P,), jnp.int32).at[p_a].set(tok)
    row_w = jnp.zeros((P,), jnp.float32).at[p_a].set(tw.reshape(-1))
    pos = p_a.reshape(T, K)
    total = cpc[-1]
    starts = jnp.arange(NT, dtype=jnp.int32) * BLK
    tile_e = jnp.minimum(
        jnp.sum((cpc[None, :] <= starts[:, None]).astype(jnp.int32), axis=1),
        E - 1)
    tile_used = (starts < total).astype(jnp.int32)
    return row_token, row_w, pos, tile_e, tile_used


def _sc_mesh():
    return plsc.VectorSubcoreMesh(core_axis_name="c", subcore_axis_name="s",
                                  num_cores=NC)


@functools.cache
def _build_sc_gather():
    @functools.partial(
        pl.kernel,
        out_type=jax.ShapeDtypeStruct((P, D), jnp.float32),
        mesh=_sc_mesh(),
        scratch_types=[
            pltpu.VMEM((GCH, GRC), jnp.int32),
            pltpu.VMEM((ROWS_G, D), jnp.float32),
            pltpu.SemaphoreType.DMA,
        ],
    )
    def _sc_gather(x_hbm, idx_hbm, out_hbm, idx_v, rows_v, sem):
        wid = lax.axis_index("s") * NC + lax.axis_index("c")
        base = wid * ROWS_G
        for c in range(GCH):
            pltpu.sync_copy(idx_hbm.at[pl.ds(base + c * GRC, GRC)],
                            idx_v.at[c])
        copies = []
        for c in range(GCH):
            copies.append(
                pltpu.async_copy(x_hbm.at[idx_v.at[c]],
                                 rows_v.at[pl.ds(c * GRC, GRC)], sem))
        for cp in copies:
            cp.wait()
        pltpu.sync_copy(rows_v, out_hbm.at[pl.ds(base, ROWS_G)])

    return _sc_gather


def _ffn_body(te_ref, used_ref, xs_ref, w1_ref, w2_ref, rw_ref, y_ref):
    j = pl.program_id(0)

    @pl.when(used_ref[j] == 1)
    def _():
        xb = xs_ref[...]
        y = jnp.zeros((BLK, D), jnp.float32)
        for f in range(F // FC):
            w1c = w1_ref[0, :, f * FC:(f + 1) * FC]
            h = jnp.dot(xb, w1c, preferred_element_type=jnp.float32)
            h = jax.nn.gelu(h)
            w2c = w2_ref[0, f * FC:(f + 1) * FC, :]
            y = y + jnp.dot(h, w2c, preferred_element_type=jnp.float32)
        y_ref[...] = y * rw_ref[:, 0:1]

    @pl.when(used_ref[j] == 0)
    def _():
        y_ref[...] = jnp.zeros((BLK, D), jnp.float32)


def _ffn(tile_e, tile_used, x_sorted, w1, w2, rw_b):
    grid_spec = pltpu.PrefetchScalarGridSpec(
        num_scalar_prefetch=2,
        grid=(NT,),
        in_specs=[
            pl.BlockSpec((BLK, D), lambda j, te, us: (j, 0)),
            pl.BlockSpec((1, D, F), lambda j, te, us: (te[j], 0, 0)),
            pl.BlockSpec((1, F, D), lambda j, te, us: (te[j], 0, 0)),
            pl.BlockSpec((BLK, 128), lambda j, te, us: (j, 0)),
        ],
        out_specs=pl.BlockSpec((BLK, D), lambda j, te, us: (j, 0)),
    )
    return pl.pallas_call(
        _ffn_body,
        grid_spec=grid_spec,
        out_shape=jax.ShapeDtypeStruct((P, D), jnp.float32),
        compiler_params=pltpu.CompilerParams(
            dimension_semantics=("arbitrary",)),
    )(tile_e, tile_used, x_sorted, w1, w2, rw_b)


@functools.cache
def _build_sc_combine():
    @functools.partial(
        pl.kernel,
        out_type=jax.ShapeDtypeStruct((T, D), jnp.float32),
        mesh=_sc_mesh(),
        scratch_types=[
            pltpu.VMEM((TOK_C,), jnp.int32),
            pltpu.VMEM((TOK_C,), jnp.int32),
            pltpu.VMEM((TOK_C, D), jnp.float32),
            pltpu.VMEM((TOK_C, D), jnp.float32),
            pltpu.SemaphoreType.DMA,
        ],
    )
    def _sc_combine(y_hbm, pos0_hbm, pos1_hbm, out_hbm,
                    i0_v, i1_v, b0, b1, sem):
        wid = lax.axis_index("s") * NC + lax.axis_index("c")
        base = wid * TOK_C
        pltpu.sync_copy(pos0_hbm.at[pl.ds(base, TOK_C)], i0_v)
        pltpu.sync_copy(pos1_hbm.at[pl.ds(base, TOK_C)], i1_v)
        c0 = pltpu.async_copy(y_hbm.at[i0_v], b0, sem)
        c1 = pltpu.async_copy(y_hbm.at[i1_v], b1, sem)
        c0.wait()
        c1.wait()

        def row_body(r, carry):
            for c in range(D // L):
                sl = pl.ds(c * L, L)
                b0[r, sl] = b0[r, sl] + b1[r, sl]
            return carry

        lax.fori_loop(0, TOK_C, row_body, 0)
        pltpu.sync_copy(b0, out_hbm.at[pl.ds(base, TOK_C)])

    return _sc_combine


def kernel(x, Wg, W1, W2):
    flat_x = x.reshape(T, D)
    wg_pad = jnp.zeros((128, D), jnp.float32).at[:E].set(Wg)
    tw, ti, aux = _router(flat_x, wg_pad)
    row_token, row_w, pos, tile_e, tile_used = _dispatch_meta(ti, tw)
    rw_b = jnp.broadcast_to(row_w[:, None], (P, 128))
    x_sorted = _build_sc_gather()(flat_x, row_token)
    y = _ffn(tile_e, tile_used, x_sorted, W1, W2, rw_b)
    out = _build_sc_combine()(y, pos[:, 0], pos[:, 1])
    return out.reshape(B, T, D), aux.reshape(())


# speedup vs baseline: 1.3607x; 1.3292x over previous
import functools

import jax
import jax.numpy as jnp
from jax import lax
from jax.experimental import pallas as pl
from jax.experimental.pallas import tpu as pltpu
from jax.experimental.pallas import tpu_sc as plsc

B = 1
T = 2048
D = 768
E = 8
K = 2
F = 3072
FC = 768

A = T * K
BLK = 128
P = 5120
NT = P // BLK
NC, NS, L = 2, 16, 16
NW = NC * NS
ROWS_G = P // NW
GCH = 10
GRC = ROWS_G // GCH
TOK_C = T // NW
NEG = -1e30


def _router_body(x_ref, wg_ref, tw_ref, ti_ref, aux_ref):
    x = x_ref[...]
    wg = wg_ref[...]
    logits = lax.dot_general(x, wg, (((1,), (1,)), ((), ())),
                             preferred_element_type=jnp.float32)
    col = lax.broadcasted_iota(jnp.int32, (T, 128), 1)
    lg = jnp.where(col < E, logits, NEG)
    m1 = jnp.max(lg, axis=1, keepdims=True)
    i1 = jnp.min(jnp.where(lg == m1, col, 128), axis=1, keepdims=True)
    lg2 = jnp.where(col == i1, NEG, lg)
    m2 = jnp.max(lg2, axis=1, keepdims=True)
    i2 = jnp.min(jnp.where(lg2 == m2, col, 128), axis=1, keepdims=True)
    s = jnp.exp(m2 - m1)
    w1 = 1.0 / (1.0 + s)
    tw_ref[...] = jnp.concatenate([w1, s * w1], axis=1)
    ti_ref[...] = jnp.concatenate([i1, i2], axis=1)
    probs = jnp.exp(lg - m1)
    probs = probs / jnp.sum(probs, axis=1, keepdims=True)
    mean_p = jnp.sum(probs, axis=0) * jnp.float32(1.0 / T)
    aux_ref[0, 0] = jnp.sum(mean_p * mean_p) * jnp.float32(E)


def _router(flat_x, wg_pad):
    return pl.pallas_call(
        _router_body,
        out_shape=(
            jax.ShapeDtypeStruct((T, K), jnp.float32),
            jax.ShapeDtypeStruct((T, K), jnp.int32),
            jax.ShapeDtypeStruct((1, 1), jnp.float32),
        ),
        out_specs=(
            pl.BlockSpec(memory_space=pltpu.VMEM),
            pl.BlockSpec(memory_space=pltpu.VMEM),
            pl.BlockSpec(memory_space=pltpu.SMEM),
        ),
    )(flat_x, wg_pad)


def _dispatch_meta(ti, tw):
    e_flat = ti.reshape(-1)
    oh = (e_flat[:, None] == jnp.arange(E, dtype=jnp.int32)[None, :])
    cum = jnp.cumsum(oh.astype(jnp.int32), axis=0)
    counts = cum[-1]
    rank = jnp.take_along_axis(cum, e_flat[:, None], axis=1)[:, 0] - 1
    pc = ((counts + BLK - 1) // BLK) * BLK
    cpc = jnp.cumsum(pc)
    off = cpc - pc
    p_a = (off[e_flat] + rank).astype(jnp.int32)
    row_w = jnp.zeros((P,), jnp.float32).at[p_a].set(tw.reshape(-1))
    pos = p_a.reshape(T, K)
    total = cpc[-1]
    starts = jnp.arange(NT, dtype=jnp.int32) * BLK
    tile_e = jnp.minimum(
        jnp.sum((cpc[None, :] <= starts[:, None]).astype(jnp.int32), axis=1),
        E - 1)
    tile_used = (starts < total).astype(jnp.int32)
    return row_w, pos, tile_e, tile_used


def _sc_mesh():
    return plsc.VectorSubcoreMesh(core_axis_name="c", subcore_axis_name="s",
                                  num_cores=NC)


@functools.cache
def _build_sc_dispatch():
    @functools.partial(
        pl.kernel,
        out_type=jax.ShapeDtypeStruct((P, D), jnp.float32),
        mesh=_sc_mesh(),
        scratch_types=[
            pltpu.VMEM((TOK_C,), jnp.int32),
            pltpu.VMEM((TOK_C,), jnp.int32),
            pltpu.VMEM((TOK_C, D), jnp.float32),
            pltpu.SemaphoreType.DMA,
        ],
    )
    def _sc_dispatch(x_hbm, pos0_hbm, pos1_hbm, out_hbm,
                     i0_v, i1_v, rows_v, sem):
        wid = lax.axis_index("s") * NC + lax.axis_index("c")
        base = wid * TOK_C
        pltpu.sync_copy(pos0_hbm.at[pl.ds(base, TOK_C)], i0_v)
        pltpu.sync_copy(pos1_hbm.at[pl.ds(base, TOK_C)], i1_v)
        pltpu.sync_copy(x_hbm.at[pl.ds(base, TOK_C)], rows_v)
        c0 = pltpu.async_copy(rows_v, out_hbm.at[i0_v], sem)
        c1 = pltpu.async_copy(rows_v, out_hbm.at[i1_v], sem)
        c0.wait()
        c1.wait()

    return _sc_dispatch


def _ffn_body(te_ref, used_ref, xs_ref, w1_ref, w2_ref, rw_ref, y_ref):
    j = pl.program_id(0)

    @pl.when(used_ref[j] == 1)
    def _():
        xb = xs_ref[...]
        y = jnp.zeros((BLK, D), jnp.float32)
        for f in range(F // FC):
            w1c = w1_ref[0, :, f * FC:(f + 1) * FC]
            h = jnp.dot(xb, w1c, preferred_element_type=jnp.float32)
            h = jax.nn.gelu(h)
            w2c = w2_ref[0, f * FC:(f + 1) * FC, :]
            y = y + jnp.dot(h, w2c, preferred_element_type=jnp.float32)
        y_ref[...] = y * rw_ref[:, 0:1]

    @pl.when(used_ref[j] == 0)
    def _():
        y_ref[...] = jnp.zeros((BLK, D), jnp.float32)


def _ffn(tile_e, tile_used, x_sorted, w1, w2, rw_b):
    grid_spec = pltpu.PrefetchScalarGridSpec(
        num_scalar_prefetch=2,
        grid=(NT,),
        in_specs=[
            pl.BlockSpec((BLK, D), lambda j, te, us: (j, 0)),
            pl.BlockSpec((1, D, F), lambda j, te, us: (te[j], 0, 0)),
            pl.BlockSpec((1, F, D), lambda j, te, us: (te[j], 0, 0)),
            pl.BlockSpec((BLK, 128), lambda j, te, us: (j, 0)),
        ],
        out_specs=pl.BlockSpec((BLK, D), lambda j, te, us: (j, 0)),
    )
    return pl.pallas_call(
        _ffn_body,
        grid_spec=grid_spec,
        out_shape=jax.ShapeDtypeStruct((P, D), jnp.float32),
        compiler_params=pltpu.CompilerParams(
            dimension_semantics=("arbitrary",)),
    )(tile_e, tile_used, x_sorted, w1, w2, rw_b)


@functools.cache
def _build_sc_combine():
    @functools.partial(
        pl.kernel,
        out_type=jax.ShapeDtypeStruct((T, D), jnp.float32),
        mesh=_sc_mesh(),
        scratch_types=[
            pltpu.VMEM((TOK_C,), jnp.int32),
            pltpu.VMEM((TOK_C,), jnp.int32),
            pltpu.VMEM((TOK_C, D), jnp.float32),
            pltpu.VMEM((TOK_C, D), jnp.float32),
            pltpu.SemaphoreType.DMA,
        ],
    )
    def _sc_combine(y_hbm, pos0_hbm, pos1_hbm, out_hbm,
                    i0_v, i1_v, b0, b1, sem):
        wid = lax.axis_index("s") * NC + lax.axis_index("c")
        base = wid * TOK_C
        pltpu.sync_copy(pos0_hbm.at[pl.ds(base, TOK_C)], i0_v)
        pltpu.sync_copy(pos1_hbm.at[pl.ds(base, TOK_C)], i1_v)
        c0 = pltpu.async_copy(y_hbm.at[i0_v], b0, sem)
        c1 = pltpu.async_copy(y_hbm.at[i1_v], b1, sem)
        c0.wait()
        c1.wait()

        def row_body(r, carry):
            for c in range(D // L):
                sl = pl.ds(c * L, L)
                b0[r, sl] = b0[r, sl] + b1[r, sl]
            return carry

        lax.fori_loop(0, TOK_C, row_body, 0)
        pltpu.sync_copy(b0, out_hbm.at[pl.ds(base, TOK_C)])

    return _sc_combine


def kernel(x, Wg, W1, W2):
    flat_x = x.reshape(T, D)
    wg_pad = jnp.zeros((128, D), jnp.float32).at[:E].set(Wg)
    tw, ti, aux = _router(flat_x, wg_pad)
    row_w, pos, tile_e, tile_used = _dispatch_meta(ti, tw)
    rw_b = jnp.broadcast_to(row_w[:, None], (P, 128))
    x_sorted = _build_sc_dispatch()(flat_x, pos[:, 0], pos[:, 1])
    y = _ffn(tile_e, tile_used, x_sorted, W1, W2, rw_b)
    out = _build_sc_combine()(y, pos[:, 0], pos[:, 1])
    return out.reshape(B, T, D), aux.reshape(())


# speedup vs baseline: 1.5192x; 1.1165x over previous
import functools

import jax
import jax.numpy as jnp
from jax import lax
from jax.experimental import pallas as pl
from jax.experimental.pallas import tpu as pltpu
from jax.experimental.pallas import tpu_sc as plsc

B = 1
T = 2048
D = 768
E = 8
K = 2
F = 3072
FC = 768

A = T * K
BLK = 256
P = 6144
NT = P // BLK
NC, NS, L = 2, 16, 16
NW = NC * NS
ROWS_G = P // NW
GCH = 10
GRC = ROWS_G // GCH
TOK_C = T // NW
NEG = -1e30


def _router_body(x_ref, wg_ref, tw_ref, ti_ref, aux_ref):
    x = x_ref[...]
    wg = wg_ref[...]
    logits = lax.dot_general(x, wg, (((1,), (1,)), ((), ())),
                             preferred_element_type=jnp.float32)
    col = lax.broadcasted_iota(jnp.int32, (T, 128), 1)
    lg = jnp.where(col < E, logits, NEG)
    m1 = jnp.max(lg, axis=1, keepdims=True)
    i1 = jnp.min(jnp.where(lg == m1, col, 128), axis=1, keepdims=True)
    lg2 = jnp.where(col == i1, NEG, lg)
    m2 = jnp.max(lg2, axis=1, keepdims=True)
    i2 = jnp.min(jnp.where(lg2 == m2, col, 128), axis=1, keepdims=True)
    s = jnp.exp(m2 - m1)
    w1 = 1.0 / (1.0 + s)
    tw_ref[...] = jnp.concatenate([w1, s * w1], axis=1)
    ti_ref[...] = jnp.concatenate([i1, i2], axis=1)
    probs = jnp.exp(lg - m1)
    probs = probs / jnp.sum(probs, axis=1, keepdims=True)
    mean_p = jnp.sum(probs, axis=0) * jnp.float32(1.0 / T)
    aux_ref[0, 0] = jnp.sum(mean_p * mean_p) * jnp.float32(E)


def _router(flat_x, wg_pad):
    return pl.pallas_call(
        _router_body,
        out_shape=(
            jax.ShapeDtypeStruct((T, K), jnp.float32),
            jax.ShapeDtypeStruct((T, K), jnp.int32),
            jax.ShapeDtypeStruct((1, 1), jnp.float32),
        ),
        out_specs=(
            pl.BlockSpec(memory_space=pltpu.VMEM),
            pl.BlockSpec(memory_space=pltpu.VMEM),
            pl.BlockSpec(memory_space=pltpu.SMEM),
        ),
    )(flat_x, wg_pad)


def _dispatch_meta(ti, tw):
    e_flat = ti.reshape(-1)
    oh = (e_flat[:, None] == jnp.arange(E, dtype=jnp.int32)[None, :])
    cum = jnp.cumsum(oh.astype(jnp.int32), axis=0)
    counts = cum[-1]
    rank = jnp.sum(cum * oh.astype(jnp.int32), axis=1) - 1
    pc = ((counts + BLK - 1) // BLK) * BLK
    cpc = jnp.cumsum(pc)
    off = cpc - pc
    p_a = (off[e_flat] + rank).astype(jnp.int32)
    row_w = jnp.zeros((P,), jnp.float32).at[p_a].set(tw.reshape(-1))
    pos = p_a.reshape(T, K)
    total = cpc[-1]
    starts = jnp.arange(NT, dtype=jnp.int32) * BLK
    tile_e = jnp.minimum(
        jnp.sum((cpc[None, :] <= starts[:, None]).astype(jnp.int32), axis=1),
        E - 1)
    tile_used = (starts < total).astype(jnp.int32)
    return row_w, pos, tile_e, tile_used


def _sc_mesh():
    return plsc.VectorSubcoreMesh(core_axis_name="c", subcore_axis_name="s",
                                  num_cores=NC)


@functools.cache
def _build_sc_dispatch():
    @functools.partial(
        pl.kernel,
        out_type=jax.ShapeDtypeStruct((P, D), jnp.float32),
        mesh=_sc_mesh(),
        scratch_types=[
            pltpu.VMEM((TOK_C,), jnp.int32),
            pltpu.VMEM((TOK_C,), jnp.int32),
            pltpu.VMEM((TOK_C, D), jnp.float32),
            pltpu.SemaphoreType.DMA,
        ],
    )
    def _sc_dispatch(x_hbm, pos0_hbm, pos1_hbm, out_hbm,
                     i0_v, i1_v, rows_v, sem):
        wid = lax.axis_index("s") * NC + lax.axis_index("c")
        base = wid * TOK_C
        pltpu.sync_copy(pos0_hbm.at[pl.ds(base, TOK_C)], i0_v)
        pltpu.sync_copy(pos1_hbm.at[pl.ds(base, TOK_C)], i1_v)
        pltpu.sync_copy(x_hbm.at[pl.ds(base, TOK_C)], rows_v)
        c0 = pltpu.async_copy(rows_v, out_hbm.at[i0_v], sem)
        c1 = pltpu.async_copy(rows_v, out_hbm.at[i1_v], sem)
        c0.wait()
        c1.wait()

    return _sc_dispatch


def _ffn_body(te_ref, used_ref, xs_ref, w1_ref, w2_ref, rw_ref, y_ref):
    j = pl.program_id(0)

    @pl.when(used_ref[j] == 1)
    def _():
        xb = xs_ref[...]
        y = jnp.zeros((BLK, D), jnp.float32)
        for f in range(F // FC):
            w1c = w1_ref[0, :, f * FC:(f + 1) * FC]
            h = jnp.dot(xb, w1c, preferred_element_type=jnp.float32)
            h = jax.nn.gelu(h)
            w2c = w2_ref[0, f * FC:(f + 1) * FC, :]
            y = y + jnp.dot(h, w2c, preferred_element_type=jnp.float32)
        y_ref[...] = y * rw_ref[:, 0:1]

    @pl.when(used_ref[j] == 0)
    def _():
        y_ref[...] = jnp.zeros((BLK, D), jnp.float32)


def _ffn(tile_e, tile_used, x_sorted, w1, w2, rw_b):
    grid_spec = pltpu.PrefetchScalarGridSpec(
        num_scalar_prefetch=2,
        grid=(NT,),
        in_specs=[
            pl.BlockSpec((BLK, D), lambda j, te, us: (j, 0)),
            pl.BlockSpec((1, D, F), lambda j, te, us: (te[j], 0, 0)),
            pl.BlockSpec((1, F, D), lambda j, te, us: (te[j], 0, 0)),
            pl.BlockSpec((BLK, 128), lambda j, te, us: (j, 0)),
        ],
        out_specs=pl.BlockSpec((BLK, D), lambda j, te, us: (j, 0)),
    )
    return pl.pallas_call(
        _ffn_body,
        grid_spec=grid_spec,
        out_shape=jax.ShapeDtypeStruct((P, D), jnp.float32),
        compiler_params=pltpu.CompilerParams(
            dimension_semantics=("arbitrary",)),
    )(tile_e, tile_used, x_sorted, w1, w2, rw_b)


@functools.cache
def _build_sc_combine():
    @functools.partial(
        pl.kernel,
        out_type=jax.ShapeDtypeStruct((T, D), jnp.float32),
        mesh=_sc_mesh(),
        scratch_types=[
            pltpu.VMEM((TOK_C,), jnp.int32),
            pltpu.VMEM((TOK_C,), jnp.int32),
            pltpu.VMEM((TOK_C, D), jnp.float32),
            pltpu.VMEM((TOK_C, D), jnp.float32),
            pltpu.SemaphoreType.DMA,
        ],
    )
    def _sc_combine(y_hbm, pos0_hbm, pos1_hbm, out_hbm,
                    i0_v, i1_v, b0, b1, sem):
        wid = lax.axis_index("s") * NC + lax.axis_index("c")
        base = wid * TOK_C
        pltpu.sync_copy(pos0_hbm.at[pl.ds(base, TOK_C)], i0_v)
        pltpu.sync_copy(pos1_hbm.at[pl.ds(base, TOK_C)], i1_v)
        c0 = pltpu.async_copy(y_hbm.at[i0_v], b0, sem)
        c1 = pltpu.async_copy(y_hbm.at[i1_v], b1, sem)
        c0.wait()
        c1.wait()

        def row_body(r, carry):
            for c in range(D // L):
                sl = pl.ds(c * L, L)
                b0[r, sl] = b0[r, sl] + b1[r, sl]
            return carry

        lax.fori_loop(0, TOK_C, row_body, 0)
        pltpu.sync_copy(b0, out_hbm.at[pl.ds(base, TOK_C)])

    return _sc_combine


def kernel(x, Wg, W1, W2):
    flat_x = x.reshape(T, D)
    wg_pad = jnp.zeros((128, D), jnp.float32).at[:E].set(Wg)
    tw, ti, aux = _router(flat_x, wg_pad)
    row_w, pos, tile_e, tile_used = _dispatch_meta(ti, tw)
    rw_b = jnp.broadcast_to(row_w[:, None], (P, 128))
    x_sorted = _build_sc_dispatch()(flat_x, pos[:, 0], pos[:, 1])
    y = _ffn(tile_e, tile_used, x_sorted, W1, W2, rw_b)
    out = _build_sc_combine()(y, pos[:, 0], pos[:, 1])
    return out.reshape(B, T, D), aux.reshape(())
